# baseline (speedup 1.0000x reference)
import functools
import jax
import jax.numpy as jnp
from jax import lax
from jax.experimental import pallas as pl
from jax.experimental.pallas import tpu as pltpu
from jax.experimental.pallas import tpu_sc as plsc

H = 512
W = 512
NG = 10000
MU_BORDER = 1.05
S_MIN = 1.0 / 30.0
S_MAX = 1.0 / 0.75

NC = 2
NS = 16
NW = NC * NS
NGP = 10240
GPT = NGP // NW
NGRP = GPT // 16
NRAY = 2048
RPAD = NRAY + 32
OSTR = RPAD
OSZ = 3 * OSTR
NPAR = 34


def _sc_body(gp_hbm, locnt_hbm, rays_hbm, out_hbm,
             gp_v, locnt_v, rays_v, acc_v, tmp_v, shared_v):
    cid = lax.axis_index("c")
    sid = lax.axis_index("s")
    wid = sid * NC + cid
    gbase = wid * GPT

    for r in range(NPAR):
        pltpu.sync_copy(gp_hbm.at[pl.ds(r * NGP + gbase, GPT)],
                        gp_v.at[pl.ds(r * GPT, GPT)])
    for r in range(2):
        pltpu.sync_copy(locnt_hbm.at[pl.ds(r * NGP + gbase, GPT)],
                        locnt_v.at[pl.ds(r * GPT, GPT)])
    pltpu.sync_copy(rays_hbm, rays_v)

    zero16 = jnp.zeros((16,), jnp.float32)

    def zloop(i, _):
        acc_v[pl.ds(i * 16, 16)] = zero16
        return 0
    lax.fori_loop(0, OSZ // 16, zloop, 0)

    def group(gg, _):
        gb = gg * 16
        lov = locnt_v[pl.ds(gb, 16)]
        nchv = locnt_v[pl.ds(GPT + gb, 16)]
        par16 = [gp_v[pl.ds(r * GPT + gb, 16)] for r in range(NPAR)]
        for j in range(16):
            lo_g = lov[j]
            nch_g = nchv[j]
            par = [p[j] for p in par16]
            c0s, c1s, s0s, s1s, cas, sas, ops = par[:7]
            sh = par[7:]

            def pair_math(base):
                x0 = rays_v[pl.ds(base, 16)]
                x1 = rays_v[pl.ds(RPAD + base, 16)]
                v0 = x0 - c0s
                v1 = x1 - c1s
                a0 = s0s * v0
                a1 = s1s * v1
                d2 = a0 * a0 + a1 * a1
                wgt = jnp.where(d2 < 25.0, jnp.exp(-d2), 0.0) * ops
                n2 = v0 * v0 + v1 * v1
                yi = 0x5F3759DF - (lax.bitcast_convert_type(n2, jnp.int32) >> 1)
                y = lax.bitcast_convert_type(yi, jnp.float32)
                hn = 0.5 * n2
                y = y * (1.5 - hn * y * y)
                y = y * (1.5 - hn * y * y)
                y = y * (1.5 - hn * y * y)
                inv = 1.0 / (1e-10 + n2 * y)
                vn0 = v0 * inv
                vn1 = v1 * inv
                sin1 = cas * vn0 - sas * vn1
                cos1 = sas * vn0 + cas * vn1
                sin2 = sin1 * cos1 + cos1 * sin1
                cos2 = cos1 * cos1 - sin1 * sin1
                sin3 = sin2 * cos1 + cos2 * sin1
                cos3 = cos2 * cos1 - sin2 * sin1
                sin4 = sin3 * cos1 + cos3 * sin1
                cos4 = cos3 * cos1 - sin3 * sin1
                for ch in range(3):
                    t = (sh[ch] + sin1 * sh[3 + ch] + cos1 * sh[6 + ch]
                         + sin2 * sh[9 + ch] + cos2 * sh[12 + ch]
                         + sin3 * sh[15 + ch] + cos3 * sh[18 + ch]
                         + sin4 * sh[21 + ch] + cos4 * sh[24 + ch])
                    rgb = 1.0 / (1.0 + jnp.exp(-t))
                    sl = pl.ds(ch * OSTR + base, 16)
                    acc_v[sl] = acc_v[sl] + wgt * rgb

            def chunk(i, _):
                base = lo_g + i * 32
                pair_math(base)
                pair_math(base + 16)
                return 0
            lax.fori_loop(0, nch_g, chunk, 0)
        return 0
    lax.fori_loop(0, NGRP, group, 0)

    pltpu.sync_copy(acc_v, shared_v.at[pl.ds(sid * OSZ, OSZ)])
    plsc.subcore_barrier()

    @pl.when(sid == 0)
    def _():
        for s in range(1, NS):
            pltpu.sync_copy(shared_v.at[pl.ds(s * OSZ, OSZ)], tmp_v)

            def addloop(i, _):
                sl = pl.ds(i * 16, 16)
                acc_v[sl] = acc_v[sl] + tmp_v[sl]
                return 0
            lax.fori_loop(0, OSZ // 16, addloop, 0)
        pltpu.sync_copy(acc_v, out_hbm.at[pl.ds(cid * OSZ, OSZ)])


def _splat_sc(gp, locnt, rays):
    mesh = plsc.VectorSubcoreMesh(core_axis_name="c", subcore_axis_name="s",
                                  num_cores=NC, num_subcores=NS)
    f = pl.kernel(
        _sc_body,
        out_type=jax.ShapeDtypeStruct((NC * OSZ,), jnp.float32),
        mesh=mesh,
        scratch_types=[
            pltpu.VMEM((NPAR * GPT,), jnp.float32),
            pltpu.VMEM((2 * GPT,), jnp.int32),
            pltpu.VMEM((2 * RPAD,), jnp.float32),
            pltpu.VMEM((OSZ,), jnp.float32),
            pltpu.VMEM((OSZ,), jnp.float32),
            pltpu.VMEM_SHARED((NS * OSZ,), jnp.float32),
        ],
    )
    return f(gp, locnt, rays)


def kernel(x_bx2, opacity, rgbsh, mu, scale, angle):
    f32 = jnp.float32
    i32 = jnp.int32
    xs0, xs1, ordx = lax.sort(
        (x_bx2[:, 0], x_bx2[:, 1], jnp.arange(NRAY, dtype=i32)), num_keys=1)
    sent = jnp.full((32,), 1.0e6, f32)
    rays = jnp.concatenate([xs0, sent, xs1, jnp.zeros((32,), f32)])

    gmu = jnp.tanh(mu) * MU_BORDER
    c0 = (gmu[:, 0] + 1.0) * (0.5 * W)
    c1 = (gmu[:, 1] + 1.0) * (0.5 * H)
    S = jnp.clip(scale, 0.0, 1.0) * (S_MAX - S_MIN) + S_MIN
    alpha = jnp.tanh(angle) * 3.1416
    ca = jnp.cos(alpha)
    sa = jnp.sin(alpha)
    ops = jax.nn.sigmoid(opacity)
    sh_t = rgbsh.reshape(NG, 27).T

    r0 = 5.0 / S[:, 0]
    grid = 4.0 * jnp.arange(129, dtype=f32)
    cum = jnp.sum(xs0[None, :] < grid[:, None], axis=1).astype(i32)
    blo = jnp.clip(jnp.floor((c0 - r0) * 0.25), 0, 128).astype(i32)
    bhi = jnp.clip(jnp.ceil((c0 + r0) * 0.25), 0, 128).astype(i32)
    karr = jnp.arange(129, dtype=i32)
    lo = jnp.sum(jnp.where(blo[:, None] == karr[None, :],
                           cum[None, :], 0), axis=1)
    hi = jnp.sum(jnp.where(bhi[:, None] == karr[None, :],
                           cum[None, :], 0), axis=1)
    chunk_lo = (lo // 16) * 16
    nch = (hi - chunk_lo + 31) // 32

    pad = NGP - NG
    def padf(v, val=0.0):
        return jnp.concatenate([v.astype(f32), jnp.full((pad,), val, f32)])
    gp = jnp.concatenate(
        [padf(c0), padf(c1), padf(S[:, 0], 1.0), padf(S[:, 1], 1.0),
         padf(ca), padf(sa), padf(ops)]
        + [padf(sh_t[r]) for r in range(27)])
    locnt = jnp.concatenate([
        jnp.concatenate([chunk_lo, jnp.zeros((pad,), i32)]),
        jnp.concatenate([nch, jnp.zeros((pad,), i32)]),
    ])

    out = _splat_sc(gp, locnt, rays)
    part = out.reshape(NC, 3, OSTR)[:, :, :NRAY].sum(axis=0).T
    onehot = (ordx[:, None] == jnp.arange(NRAY, dtype=i32)[None, :])
    return jnp.matmul(onehot.astype(f32).T, part,
                      precision=lax.Precision.HIGHEST)


# speedup vs baseline: 1.5077x; 1.5077x over previous
import functools
import jax
import jax.numpy as jnp
from jax import lax
from jax.experimental import pallas as pl
from jax.experimental.pallas import tpu as pltpu
from jax.experimental.pallas import tpu_sc as plsc

H = 512
W = 512
NG = 10000
MU_BORDER = 1.05
S_MIN = 1.0 / 30.0
S_MAX = 1.0 / 0.75

NC = 2
NS = 16
NW = NC * NS
NGP = 10240
GPT = NGP // NW
NGRP = GPT // 16
NRAY = 2048
RPAD = NRAY + 32
OSTR = RPAD
OSZ = 3 * OSTR
NPAR = 34


def _sc_body(gp_hbm, locnt_hbm, rays_hbm, out_hbm,
             gp_v, locnt_v, rays_v, acc_v, tmp_v, shared_v):
    cid = lax.axis_index("c")
    sid = lax.axis_index("s")
    wid = sid * NC + cid
    gbase = wid * GPT

    for r in range(NPAR):
        pltpu.sync_copy(gp_hbm.at[pl.ds(r * NGP + gbase, GPT)],
                        gp_v.at[pl.ds(r * GPT, GPT)])
    for r in range(2):
        pltpu.sync_copy(locnt_hbm.at[pl.ds(r * NGP + gbase, GPT)],
                        locnt_v.at[pl.ds(r * GPT, GPT)])
    pltpu.sync_copy(rays_hbm, rays_v)

    zero16 = jnp.zeros((16,), jnp.float32)

    def zloop(i, _):
        acc_v[pl.ds(i * 16, 16)] = zero16
        return 0
    lax.fori_loop(0, OSZ // 16, zloop, 0)

    def group(gg, _):
        gb = gg * 16
        lov = locnt_v[pl.ds(gb, 16)]
        nchv = locnt_v[pl.ds(GPT + gb, 16)]
        par16 = [gp_v[pl.ds(r * GPT + gb, 16)] for r in range(NPAR)]
        for j in range(16):
            lo_g = lov[j]
            nch_g = nchv[j]
            par = [p[j] for p in par16]
            c0s, c1s, s0s, s1s, cas, sas, ops = par[:7]
            sh = par[7:]

            def pair_math(base):
                x0 = rays_v[pl.ds(base, 16)]
                x1 = rays_v[pl.ds(RPAD + base, 16)]
                v0 = x0 - c0s
                v1 = x1 - c1s
                a0 = s0s * v0
                a1 = s1s * v1
                d2 = a0 * a0 + a1 * a1
                wgt = jnp.where(d2 < 25.0, jnp.exp(-d2), 0.0) * ops
                n2 = v0 * v0 + v1 * v1
                yi = 0x5F3759DF - (lax.bitcast_convert_type(n2, jnp.int32) >> 1)
                y = lax.bitcast_convert_type(yi, jnp.float32)
                hn = 0.5 * n2
                y = y * (1.5 - hn * y * y)
                y = y * (1.5 - hn * y * y)
                y = y * (1.5 - hn * y * y)
                vn0 = v0 * y
                vn1 = v1 * y
                sin1 = cas * vn0 - sas * vn1
                cos1 = sas * vn0 + cas * vn1
                sin2 = sin1 * cos1 + cos1 * sin1
                cos2 = cos1 * cos1 - sin1 * sin1
                sin3 = sin2 * cos1 + cos2 * sin1
                cos3 = cos2 * cos1 - sin2 * sin1
                sin4 = sin3 * cos1 + cos3 * sin1
                cos4 = cos3 * cos1 - sin3 * sin1
                for ch in range(3):
                    t = (sh[ch] + sin1 * sh[3 + ch] + cos1 * sh[6 + ch]
                         + sin2 * sh[9 + ch] + cos2 * sh[12 + ch]
                         + sin3 * sh[15 + ch] + cos3 * sh[18 + ch]
                         + sin4 * sh[21 + ch] + cos4 * sh[24 + ch])
                    rgb = 1.0 / (1.0 + jnp.exp(-t))
                    sl = pl.ds(ch * OSTR + base, 16)
                    acc_v[sl] = acc_v[sl] + wgt * rgb

            def chunk(i, _):
                pair_math(lo_g + i * 16)
                return 0
            lax.fori_loop(0, nch_g, chunk, 0)
        return 0
    lax.fori_loop(0, NGRP, group, 0)

    pltpu.sync_copy(acc_v, shared_v.at[pl.ds(sid * OSZ, OSZ)])
    plsc.subcore_barrier()

    @pl.when(sid == 0)
    def _():
        for s in range(1, NS):
            pltpu.sync_copy(shared_v.at[pl.ds(s * OSZ, OSZ)], tmp_v)

            def addloop(i, _):
                sl = pl.ds(i * 16, 16)
                acc_v[sl] = acc_v[sl] + tmp_v[sl]
                return 0
            lax.fori_loop(0, OSZ // 16, addloop, 0)
        pltpu.sync_copy(acc_v, out_hbm.at[pl.ds(cid * OSZ, OSZ)])


def _splat_sc(gp, locnt, rays):
    mesh = plsc.VectorSubcoreMesh(core_axis_name="c", subcore_axis_name="s",
                                  num_cores=NC, num_subcores=NS)
    f = pl.kernel(
        _sc_body,
        out_type=jax.ShapeDtypeStruct((NC * OSZ,), jnp.float32),
        mesh=mesh,
        scratch_types=[
            pltpu.VMEM((NPAR * GPT,), jnp.float32),
            pltpu.VMEM((2 * GPT,), jnp.int32),
            pltpu.VMEM((2 * RPAD,), jnp.float32),
            pltpu.VMEM((OSZ,), jnp.float32),
            pltpu.VMEM((OSZ,), jnp.float32),
            pltpu.VMEM_SHARED((NS * OSZ,), jnp.float32),
        ],
    )
    return f(gp, locnt, rays)


def kernel(x_bx2, opacity, rgbsh, mu, scale, angle):
    f32 = jnp.float32
    i32 = jnp.int32
    xs0, xs1, ordx = lax.sort(
        (x_bx2[:, 0], x_bx2[:, 1], jnp.arange(NRAY, dtype=i32)), num_keys=1)
    sent = jnp.full((32,), 1.0e6, f32)
    rays = jnp.concatenate([xs0, sent, xs1, jnp.zeros((32,), f32)])

    gmu = jnp.tanh(mu) * MU_BORDER
    c0 = (gmu[:, 0] + 1.0) * (0.5 * W)
    c1 = (gmu[:, 1] + 1.0) * (0.5 * H)
    S = jnp.clip(scale, 0.0, 1.0) * (S_MAX - S_MIN) + S_MIN
    alpha = jnp.tanh(angle) * 3.1416
    ca = jnp.cos(alpha)
    sa = jnp.sin(alpha)
    ops = jax.nn.sigmoid(opacity)
    sh_t = rgbsh.reshape(NG, 27).T

    r0 = 5.0 / S[:, 0]
    grid = 4.0 * jnp.arange(129, dtype=f32)
    cum = jnp.sum(xs0[None, :] < grid[:, None], axis=1).astype(i32)
    blo = jnp.clip(jnp.floor((c0 - r0) * 0.25), 0, 128).astype(i32)
    bhi = jnp.clip(jnp.ceil((c0 + r0) * 0.25), 0, 128).astype(i32)
    karr = jnp.arange(129, dtype=i32)
    lo = jnp.sum(jnp.where(blo[:, None] == karr[None, :],
                           cum[None, :], 0), axis=1)
    hi = jnp.sum(jnp.where(bhi[:, None] == karr[None, :],
                           cum[None, :], 0), axis=1)
    chunk_lo = (lo // 16) * 16
    nch = (hi - chunk_lo + 15) // 16

    pad = NGP - NG
    def padf(v, val=0.0):
        return jnp.concatenate([v.astype(f32), jnp.full((pad,), val, f32)])
    gp = jnp.concatenate(
        [padf(c0), padf(c1), padf(S[:, 0], 1.0), padf(S[:, 1], 1.0),
         padf(ca), padf(sa), padf(ops)]
        + [padf(sh_t[r]) for r in range(27)])
    locnt = jnp.concatenate([
        jnp.concatenate([chunk_lo, jnp.zeros((pad,), i32)]),
        jnp.concatenate([nch, jnp.zeros((pad,), i32)]),
    ])

    out = _splat_sc(gp, locnt, rays)
    part = out.reshape(NC, 3, OSTR)[:, :, :NRAY].sum(axis=0).T
    onehot = (ordx[:, None] == jnp.arange(NRAY, dtype=i32)[None, :])
    return jnp.matmul(onehot.astype(f32).T, part,
                      precision=lax.Precision.HIGHEST)
